# baseline (speedup 1.0000x reference)
import functools

import jax
import jax.numpy as jnp
from jax import lax
from jax.experimental import pallas as pl
from jax.experimental.pallas import tpu as pltpu
from jax.experimental.pallas import tpu_sc as plsc

B = 4096
F = 26
V = 100000
D = 16

NC = 2
NS = 16
NW = NC * NS
BPW = B // NW
NCHUNK = BPW // 16
NPOS = BPW * F // 16


def _sc_body(x_hbm, e2_hbm, w2_hbm, lin_hbm, parts_hbm,
             x_v, idx_v, rows_v, wv_v, out_v, part_v, sem_e, sem_w):
    c = lax.axis_index("c")
    s = lax.axis_index("s")
    wid = s * NC + c
    base = wid * BPW

    pltpu.sync_copy(x_hbm.at[pl.ds(base * F, BPW * F)], x_v)

    iota = lax.iota(jnp.int32, 16)
    for ch in range(NPOS):
        fs = (ch * 16) % F
        bs = (ch * 16) // F
        v = x_v[pl.ds(ch * 16, 16)]
        f_vec = fs + iota
        wrap = f_vec >= F
        f_vec = jnp.where(wrap, f_vec - F, f_vec)
        b_vec = jnp.where(wrap, bs + 1, bs)
        plsc.store_scatter(idx_v, [f_vec, b_vec], v + V * f_vec)

    cps_e, cps_w = [], []
    for f in range(F):
        cps_e.append(pltpu.async_copy(e2_hbm.at[idx_v.at[f]], rows_v.at[f], sem_e))
        cps_w.append(pltpu.async_copy(w2_hbm.at[idx_v.at[f]], wv_v.at[f], sem_w))
    for cp in cps_e:
        cp.wait()

    def body(bb, carry):
        p_acc, q_acc = carry
        e0 = rows_v[0, bb, :]
        s_v = e0
        q_v = e0 * e0
        for f in range(1, F):
            e = rows_v[f, bb, :]
            s_v = s_v + e
            q_v = q_v + e * e
        return (p_acc + s_v * s_v, q_acc + q_v)

    zero = jnp.zeros((16,), jnp.float32)
    p_acc, q_acc = lax.fori_loop(0, BPW, body, (zero, zero))
    part_v[...] = p_acc - q_acc
    pltpu.sync_copy(part_v, parts_hbm.at[wid])

    for cp in cps_w:
        cp.wait()
    for ci in range(NCHUNK):
        acc = wv_v[0, pl.ds(ci * 16, 16)]
        for f in range(1, F):
            acc = acc + wv_v[f, pl.ds(ci * 16, 16)]
        out_v[pl.ds(ci * 16, 16)] = acc
    pltpu.sync_copy(out_v, lin_hbm.at[pl.ds(base, BPW)])


@functools.partial(
    pl.kernel,
    out_type=(
        jax.ShapeDtypeStruct((B,), jnp.float32),
        jax.ShapeDtypeStruct((NW, 16), jnp.float32),
    ),
    mesh=plsc.VectorSubcoreMesh(core_axis_name="c", subcore_axis_name="s"),
    compiler_params=pltpu.CompilerParams(
        use_tc_tiling_on_sc=False, needs_layout_passes=False),
    scratch_types=[
        pltpu.VMEM((BPW * F,), jnp.int32),
        pltpu.VMEM((F, BPW), jnp.int32),
        pltpu.VMEM((F, BPW, D), jnp.float32),
        pltpu.VMEM((F, BPW), jnp.float32),
        pltpu.VMEM((BPW,), jnp.float32),
        pltpu.VMEM((16,), jnp.float32),
        pltpu.SemaphoreType.DMA,
        pltpu.SemaphoreType.DMA,
    ],
)
def _sc_gather_fm(x_hbm, e2_hbm, w2_hbm, lin_hbm, parts_hbm,
                  x_v, idx_v, rows_v, wv_v, out_v, part_v, sem_e, sem_w):
    _sc_body(x_hbm, e2_hbm, w2_hbm, lin_hbm, parts_hbm,
             x_v, idx_v, rows_v, wv_v, out_v, part_v, sem_e, sem_w)


def _tc_combine(lin_ref, parts_ref, b_ref, out_ref):
    inter = 0.5 * jnp.sum(parts_ref[...]) + b_ref[0]
    out_ref[...] = lin_ref[...] + inter


def kernel(inputs, W_lin, b, E):
    x_flat = inputs.astype(jnp.int32).reshape(B * F)
    e2 = E.reshape(F * V, D)
    w2 = W_lin.reshape(F * V)

    lin, parts = _sc_gather_fm(x_flat, e2, w2)

    out = pl.pallas_call(
        _tc_combine,
        out_shape=jax.ShapeDtypeStruct((B,), jnp.float32),
    )(lin, parts, b)
    return out[:, None]


# speedup vs baseline: 1.0017x; 1.0017x over previous
import functools

import jax
import jax.numpy as jnp
from jax import lax
from jax.experimental import pallas as pl
from jax.experimental.pallas import tpu as pltpu
from jax.experimental.pallas import tpu_sc as plsc

B = 4096
F = 26
V = 100000
D = 16

NC = 2
NS = 16
NW = NC * NS
BPW = B // NW
NCHUNK = BPW // 16
NPOS = BPW * F // 16


def _sc_body(x_hbm, e2_hbm, w2_hbm, lin_hbm, parts_hbm,
             x_v, idx_v, rows_v, wv_v, out_v, part_v, sem_e, sem_w):
    c = lax.axis_index("c")
    s = lax.axis_index("s")
    wid = s * NC + c
    base = wid * BPW

    pltpu.sync_copy(x_hbm.at[pl.ds(base * F, BPW * F)], x_v)

    iota = lax.iota(jnp.int32, 16)
    for ch in range(NPOS):
        fs = (ch * 16) % F
        bs = (ch * 16) // F
        v = x_v[pl.ds(ch * 16, 16)]
        f_vec = fs + iota
        wrap = f_vec >= F
        f_vec = jnp.where(wrap, f_vec - F, f_vec)
        b_vec = jnp.where(wrap, bs + 1, bs)
        plsc.store_scatter(idx_v, [f_vec, b_vec], v)

    cps_e, cps_w = [], []
    for f in range(F):
        cps_e.append(pltpu.async_copy(
            e2_hbm.at[f].at[idx_v.at[f]], rows_v.at[f], sem_e))
        cps_w.append(pltpu.async_copy(
            w2_hbm.at[f].at[idx_v.at[f]], wv_v.at[f], sem_w))
    for cp in cps_e:
        cp.wait()

    def body(bb, carry):
        p_acc, q_acc = carry
        e0 = rows_v[0, bb, :]
        s_v = e0
        q_v = e0 * e0
        for f in range(1, F):
            e = rows_v[f, bb, :]
            s_v = s_v + e
            q_v = q_v + e * e
        return (p_acc + s_v * s_v, q_acc + q_v)

    zero = jnp.zeros((16,), jnp.float32)
    p_acc, q_acc = lax.fori_loop(0, BPW, body, (zero, zero))
    part_v[...] = p_acc - q_acc
    pltpu.sync_copy(part_v, parts_hbm.at[wid])

    for cp in cps_w:
        cp.wait()
    for ci in range(NCHUNK):
        acc = wv_v[0, pl.ds(ci * 16, 16)]
        for f in range(1, F):
            acc = acc + wv_v[f, pl.ds(ci * 16, 16)]
        out_v[pl.ds(ci * 16, 16)] = acc
    pltpu.sync_copy(out_v, lin_hbm.at[pl.ds(base, BPW)])


@functools.partial(
    pl.kernel,
    out_type=(
        jax.ShapeDtypeStruct((B,), jnp.float32),
        jax.ShapeDtypeStruct((NW, 16), jnp.float32),
    ),
    mesh=plsc.VectorSubcoreMesh(core_axis_name="c", subcore_axis_name="s"),
    compiler_params=pltpu.CompilerParams(
        use_tc_tiling_on_sc=False, needs_layout_passes=False),
    scratch_types=[
        pltpu.VMEM((BPW * F,), jnp.int32),
        pltpu.VMEM((F, BPW), jnp.int32),
        pltpu.VMEM((F, BPW, D), jnp.float32),
        pltpu.VMEM((F, BPW), jnp.float32),
        pltpu.VMEM((BPW,), jnp.float32),
        pltpu.VMEM((16,), jnp.float32),
        pltpu.SemaphoreType.DMA,
        pltpu.SemaphoreType.DMA,
    ],
)
def _sc_gather_fm(x_hbm, e2_hbm, w2_hbm, lin_hbm, parts_hbm,
                  x_v, idx_v, rows_v, wv_v, out_v, part_v, sem_e, sem_w):
    _sc_body(x_hbm, e2_hbm, w2_hbm, lin_hbm, parts_hbm,
             x_v, idx_v, rows_v, wv_v, out_v, part_v, sem_e, sem_w)


def _tc_combine(lin_ref, parts_ref, b_ref, out_ref):
    inter = 0.5 * jnp.sum(parts_ref[...]) + b_ref[0]
    out_ref[...] = lin_ref[...] + inter


def kernel(inputs, W_lin, b, E):
    x_flat = inputs.astype(jnp.int32).reshape(B * F)

    lin, parts = _sc_gather_fm(x_flat, E, W_lin)

    out = pl.pallas_call(
        _tc_combine,
        out_shape=jax.ShapeDtypeStruct((B,), jnp.float32),
    )(lin, parts, b)
    return out[:, None]


# speedup vs baseline: 3.1621x; 3.1568x over previous
import functools

import jax
import jax.numpy as jnp
from jax import lax
from jax.experimental import pallas as pl
from jax.experimental.pallas import tpu as pltpu
from jax.experimental.pallas import tpu_sc as plsc

B = 4096
F = 26
V = 100000
D = 16

NC = 2
NS = 16
NW = NC * NS
BPW = B // NW
NCHUNK = BPW // 16


def _sc_body(xt_hbm, et_hbm, w_hbm, lin_hbm, parts_hbm,
             idx_v, ebuf_v, wv_v, out_v, part_v, sem_e, sem_w):
    c = lax.axis_index("c")
    s = lax.axis_index("s")
    wid = s * NC + c
    base = wid * BPW

    pltpu.sync_copy(xt_hbm.at[:, pl.ds(base, BPW)], idx_v)

    def fire(f, carry):
        row = idx_v.at[f]
        pltpu.async_copy(w_hbm.at[f].at[row], wv_v.at[f], sem_w)
        for d in range(D):
            pltpu.async_copy(et_hbm.at[f].at[d].at[row], ebuf_v.at[f].at[d],
                             sem_e)
        return carry

    lax.fori_loop(0, F, fire, 0)

    pltpu.make_async_copy(et_hbm.at[:, :, pl.ds(0, BPW)], ebuf_v, sem_e).wait()

    zero = jnp.zeros((16,), jnp.float32)

    def body(d, carry):
        out = []
        for ci in range(NCHUNK):
            sl = pl.ds(ci * 16, 16)
            e0 = ebuf_v[0, d, sl]
            s_v = e0
            q_v = e0 * e0
            for f in range(1, F):
                e = ebuf_v[f, d, sl]
                s_v = s_v + e
                q_v = q_v + e * e
            out.append(carry[ci] + s_v * s_v - q_v)
        return tuple(out)

    parts = lax.fori_loop(0, D, body, (zero,) * NCHUNK)
    acc = parts[0]
    for ci in range(1, NCHUNK):
        acc = acc + parts[ci]
    part_v[...] = acc
    pltpu.sync_copy(part_v, parts_hbm.at[wid])

    pltpu.make_async_copy(w_hbm.at[:, pl.ds(0, BPW)], wv_v, sem_w).wait()
    for ci in range(NCHUNK):
        sl = pl.ds(ci * 16, 16)
        accw = wv_v[0, sl]
        for f in range(1, F):
            accw = accw + wv_v[f, sl]
        out_v[sl] = accw
    pltpu.sync_copy(out_v, lin_hbm.at[pl.ds(base, BPW)])


@functools.partial(
    pl.kernel,
    out_type=(
        jax.ShapeDtypeStruct((B,), jnp.float32),
        jax.ShapeDtypeStruct((NW, 16), jnp.float32),
    ),
    mesh=plsc.VectorSubcoreMesh(core_axis_name="c", subcore_axis_name="s"),
    compiler_params=pltpu.CompilerParams(
        use_tc_tiling_on_sc=False, needs_layout_passes=False),
    scratch_types=[
        pltpu.VMEM((F, BPW), jnp.int32),
        pltpu.VMEM((F, D, BPW), jnp.float32),
        pltpu.VMEM((F, BPW), jnp.float32),
        pltpu.VMEM((BPW,), jnp.float32),
        pltpu.VMEM((16,), jnp.float32),
        pltpu.SemaphoreType.DMA,
        pltpu.SemaphoreType.DMA,
    ],
)
def _sc_gather_fm(xt_hbm, et_hbm, w_hbm, lin_hbm, parts_hbm,
                  idx_v, ebuf_v, wv_v, out_v, part_v, sem_e, sem_w):
    _sc_body(xt_hbm, et_hbm, w_hbm, lin_hbm, parts_hbm,
             idx_v, ebuf_v, wv_v, out_v, part_v, sem_e, sem_w)


def _tc_combine(lin_ref, parts_ref, b_ref, out_ref):
    inter = 0.5 * jnp.sum(parts_ref[...]) + b_ref[0]
    out_ref[...] = lin_ref[...] + inter


def kernel(inputs, W_lin, b, E):
    x_t = inputs.astype(jnp.int32).T
    e_t = E.transpose(0, 2, 1)

    lin, parts = _sc_gather_fm(x_t, e_t, W_lin)

    out = pl.pallas_call(
        _tc_combine,
        out_shape=jax.ShapeDtypeStruct((B,), jnp.float32),
    )(lin, parts, b)
    return out[:, None]


# speedup vs baseline: 4.3263x; 1.3682x over previous
import functools

import jax
import jax.numpy as jnp
from jax import lax
from jax.experimental import pallas as pl
from jax.experimental.pallas import tpu as pltpu
from jax.experimental.pallas import tpu_sc as plsc

B = 4096
F = 26
V = 100000
D = 16

NC = 2
NS = 16
NW = NC * NS
WIN = 3200
LASTLO = 99200
SLABLO = 96768
NCH = B // 16
TROWS = B + 128
DUMMY = B
CAP = B + 128


def _sc_body(xt_hbm, et_hbm, w_hbm, etail_hbm, wtail_hbm, tabs_hbm, parts_hbm,
             slab_v, wrow_v, xrow_v, loc_v, bid_v, hit_v, b2d_v, part_v,
             stab_sh, sem_s, sem_w, sem_x):
    c = lax.axis_index("c")
    s = lax.axis_index("s")
    wid = s * NC + c
    last = wid == NW - 1
    own_lo = jnp.where(last, LASTLO, wid * WIN)
    own_hi = jnp.where(last, V, own_lo + WIN)
    sbase = jnp.where(last, SLABLO, wid * WIN)
    iota = lax.iota(jnp.int32, 16)

    zvec = jnp.zeros((16,), jnp.float32)
    for r in range(128):
        for h in range(8):
            hit_v[r, pl.ds(h * 16, 16)] = zvec
    row0 = s * (TROWS // NS)
    pltpu.sync_copy(hit_v, stab_sh.at[pl.ds(row0, 128), :])
    pltpu.sync_copy(hit_v, stab_sh.at[pl.ds(row0 + 128, 128), :])
    pltpu.sync_copy(hit_v.at[pl.ds(0, 8), :],
                    stab_sh.at[pl.ds(row0 + 256, 8), :])
    plsc.subcore_barrier()

    def field_step(f, nil):
        pltpu.async_copy(et_hbm.at[f, :, pl.ds(sbase, WIN)],
                         slab_v.at[:, pl.ds(0, WIN)], sem_s)
        pltpu.async_copy(w_hbm.at[pl.ds(f, 1), pl.ds(sbase, WIN)],
                         wrow_v.at[:, pl.ds(0, WIN)], sem_w)

        @pl.when(last)
        def _():
            pltpu.async_copy(etail_hbm.at[f], slab_v.at[:, pl.ds(WIN, 128)],
                             sem_s)
            pltpu.async_copy(wtail_hbm.at[pl.ds(f, 1), :],
                             wrow_v.at[:, pl.ds(WIN, 128)], sem_w)

        pltpu.async_copy(xt_hbm.at[pl.ds(f, 1), :], xrow_v, sem_x).wait()

        def scan_step(i, nh):
            xv = xrow_v[0, pl.ds(i * 16, 16)]
            m = jnp.logical_and(xv >= own_lo, xv < own_hi)
            plsc.store_compressed(loc_v.at[pl.ds(nh, 16)], xv - sbase, mask=m)
            plsc.store_compressed(bid_v.at[pl.ds(nh, 16)], i * 16 + iota, mask=m)
            cnt = plsc.all_reduce_population_count(m)
            return nh + cnt[0]

        nh = lax.fori_loop(0, NCH, scan_step, jnp.int32(0))

        pltpu.make_async_copy(et_hbm.at[f, :, pl.ds(0, WIN)],
                              slab_v.at[:, pl.ds(0, WIN)], sem_s).wait()
        pltpu.make_async_copy(w_hbm.at[pl.ds(f, 1), pl.ds(0, WIN)],
                              wrow_v.at[:, pl.ds(0, WIN)], sem_w).wait()

        @pl.when(last)
        def _():
            pltpu.make_async_copy(etail_hbm.at[f],
                                  slab_v.at[:, pl.ds(WIN, 128)], sem_s).wait()
            pltpu.make_async_copy(wtail_hbm.at[pl.ds(f, 1), :],
                                  wrow_v.at[:, pl.ds(WIN, 128)], sem_w).wait()

        def batch_step(bi, q_acc):
            base = bi * 128
            for g in range(8):
                hb = base + g * 16
                hvec = hb + iota
                valid = hvec < nh
                lv = loc_v[pl.ds(hb, 16)]
                lv = jnp.where(valid, lv, 0)
                bv = bid_v[pl.ds(hb, 16)]
                bv = jnp.where(valid, bv, DUMMY)
                b2d_v[0, pl.ds(g * 16, 16)] = bv
                rows = g * 16 + iota
                for d in range(D):
                    vals = plsc.load_gather(
                        slab_v, [jnp.full((16,), d, jnp.int32), lv])
                    vals = jnp.where(valid, vals, 0.0)
                    q_acc = q_acc + vals * vals
                    plsc.store_scatter(
                        hit_v, [rows, jnp.full((16,), d, jnp.int32)], vals)
                wv = plsc.load_gather(wrow_v, [jnp.zeros((16,), jnp.int32),
                                               lv])
                wv = jnp.where(valid, wv, 0.0)
                plsc.store_scatter(
                    hit_v, [rows, jnp.full((16,), D, jnp.int32)], wv)
            pltpu.sync_copy(hit_v, stab_sh.at[b2d_v.at[0]], add=True)
            return q_acc

        nb = (nh + 127) // 128
        return lax.fori_loop(0, nb, batch_step, nil)

    q_acc = lax.fori_loop(0, F, field_step, jnp.zeros((16,), jnp.float32))
    for h in range(8):
        part_v[0, pl.ds(h * 16, 16)] = jnp.zeros((16,), jnp.float32)
    part_v[0, pl.ds(0, 16)] = q_acc
    pltpu.sync_copy(part_v, parts_hbm.at[pl.ds(wid, 1), :])

    plsc.subcore_barrier()
    pltpu.sync_copy(stab_sh.at[pl.ds(row0, TROWS // NS), :],
                    tabs_hbm.at[c].at[pl.ds(row0, TROWS // NS), :])


@functools.partial(
    pl.kernel,
    out_type=(
        jax.ShapeDtypeStruct((NC, TROWS, 128), jnp.float32),
        jax.ShapeDtypeStruct((NW, 128), jnp.float32),
    ),
    mesh=plsc.VectorSubcoreMesh(core_axis_name="c", subcore_axis_name="s"),
    compiler_params=pltpu.CompilerParams(
        use_tc_tiling_on_sc=True, needs_layout_passes=False),
    scratch_types=[
        pltpu.VMEM((D, WIN + 128), jnp.float32),
        pltpu.VMEM((1, WIN + 128), jnp.float32),
        pltpu.VMEM((1, B), jnp.int32),
        pltpu.VMEM((CAP + 16,), jnp.int32),
        pltpu.VMEM((CAP + 16,), jnp.int32),
        pltpu.VMEM((128, 128), jnp.float32),
        pltpu.VMEM((1, 128), jnp.int32),
        pltpu.VMEM((1, 128), jnp.float32),
        pltpu.VMEM_SHARED((TROWS, 128), jnp.float32),
        pltpu.SemaphoreType.DMA,
        pltpu.SemaphoreType.DMA,
        pltpu.SemaphoreType.DMA,
    ],
)
def _sc_fm(xt_hbm, et_hbm, w_hbm, etail_hbm, wtail_hbm, tabs_hbm, parts_hbm,
           slab_v, wrow_v, xrow_v, loc_v, bid_v, hit_v, b2d_v, part_v,
           stab_sh, sem_s, sem_w, sem_x):
    _sc_body(xt_hbm, et_hbm, w_hbm, etail_hbm, wtail_hbm, tabs_hbm, parts_hbm,
             slab_v, wrow_v, xrow_v, loc_v, bid_v, hit_v, b2d_v, part_v,
             stab_sh, sem_s, sem_w, sem_x)


def _tc_combine(tabs_ref, parts_ref, b_ref, out_ref):
    t = tabs_ref[0] + tabs_ref[1]
    sv = t[:B, :D]
    lin = jnp.sum(t[:B, D:2 * D], axis=1)
    q = jnp.sum(parts_ref[:, :D])
    inter = 0.5 * (jnp.sum(sv * sv) - q) + b_ref[0]
    out_ref[...] = lin + inter


def kernel(inputs, W_lin, b, E):
    x_t = inputs.astype(jnp.int32).T
    e_t = E.transpose(0, 2, 1)

    e_tail = jnp.pad(e_t[:, :, V - 32:], ((0, 0), (0, 0), (0, 96)))
    w_tail = jnp.pad(W_lin[:, V - 32:], ((0, 0), (0, 96)))
    tabs, parts = _sc_fm(x_t, e_t, W_lin, e_tail, w_tail)

    out = pl.pallas_call(
        _tc_combine,
        out_shape=jax.ShapeDtypeStruct((B,), jnp.float32),
    )(tabs, parts, b)
    return out[:, None]


# speedup vs baseline: 4.4243x; 1.0226x over previous
import functools

import jax
import jax.numpy as jnp
from jax import lax
from jax.experimental import pallas as pl
from jax.experimental.pallas import tpu as pltpu
from jax.experimental.pallas import tpu_sc as plsc

B = 4096
F = 26
V = 100000
D = 16

NC = 2
NS = 16
NW = NC * NS
WIN = 3200
LASTLO = 99200
SLABLO = 96768
NCH = B // 16
TROWS = B + 128
DUMMY = B
CAP = B + 128


CHA = 1664
CHB = WIN - CHA


def _sc_body(xt_hbm, et_hbm, w_hbm, etail_hbm, wtail_hbm, tabs_hbm, parts_hbm,
             slab_a, slab_b, wrow_v, xrow_v, pka_v, pkb_v, hit_v, b2d_v,
             part_v, stab_sh, sem_sa, sem_sb, sem_w, sem_x):
    c = lax.axis_index("c")
    s = lax.axis_index("s")
    wid = s * NC + c
    last = wid == NW - 1
    own_lo = jnp.where(last, LASTLO, wid * WIN)
    own_hi = jnp.where(last, V, own_lo + WIN)
    sbase = jnp.where(last, SLABLO, wid * WIN)
    iota = lax.iota(jnp.int32, 16)

    zvec = jnp.zeros((16,), jnp.float32)
    for r in range(96):
        for h in range(8):
            hit_v[r, pl.ds(h * 16, 16)] = zvec
    row0 = s * (TROWS // NS)
    pltpu.sync_copy(hit_v, stab_sh.at[pl.ds(row0, 96), :])
    pltpu.sync_copy(hit_v, stab_sh.at[pl.ds(row0 + 96, 96), :])
    pltpu.sync_copy(hit_v.at[pl.ds(0, 72), :],
                    stab_sh.at[pl.ds(row0 + 192, 72), :])
    plsc.subcore_barrier()

    def fire_a(f):
        pltpu.async_copy(et_hbm.at[f, :, pl.ds(sbase, CHA)], slab_a, sem_sa)

    def fire_b(f):
        pltpu.async_copy(et_hbm.at[f, :, pl.ds(sbase + CHA, CHB)],
                         slab_b.at[:, pl.ds(0, CHB)], sem_sb)

        @pl.when(last)
        def _():
            pltpu.async_copy(etail_hbm.at[f], slab_b.at[:, pl.ds(CHB, 128)],
                             sem_sb)

    def hits(slab, pk_ref, nh, woff, q_acc):
        def batch_step(bi, q_in):
            base = bi * 96
            q_b = q_in
            for g in range(6):
                hb = base + g * 16
                valid = hb + iota < nh
                pk = pk_ref[pl.ds(hb, 16)]
                lv = jnp.where(valid, pk & 8191, 0)
                bv = jnp.where(valid, jax.lax.shift_right_logical(pk, 13),
                               DUMMY)
                b2d_v[0, pl.ds(g * 16, 16)] = bv
                rows = g * 16 + iota
                for d in range(D):
                    vals = plsc.load_gather(
                        slab, [jnp.full((16,), d, jnp.int32), lv])
                    vals = jnp.where(valid, vals, 0.0)
                    q_b = q_b + vals * vals
                    plsc.store_scatter(
                        hit_v, [rows, jnp.full((16,), d, jnp.int32)], vals)
                wv = plsc.load_gather(wrow_v, [jnp.zeros((16,), jnp.int32),
                                               lv + woff])
                wv = jnp.where(valid, wv, 0.0)
                plsc.store_scatter(
                    hit_v, [rows, jnp.full((16,), D, jnp.int32)], wv)
            pltpu.sync_copy(hit_v, stab_sh.at[b2d_v.at[0]], add=True)
            return q_b

        nb = (nh + 95) // 96
        return lax.fori_loop(0, nb, batch_step, q_acc)

    fire_a(0)

    def field_step(f, q_acc):
        fire_b(f)
        pltpu.async_copy(w_hbm.at[pl.ds(f, 1), pl.ds(sbase, WIN)],
                         wrow_v.at[:, pl.ds(0, WIN)], sem_w)

        @pl.when(last)
        def _():
            pltpu.async_copy(wtail_hbm.at[pl.ds(f, 1), :],
                             wrow_v.at[:, pl.ds(WIN, 128)], sem_w)

        pltpu.async_copy(xt_hbm.at[pl.ds(f, 1), :], xrow_v, sem_x).wait()

        def scan_step(i, ns):
            nha, nhb = ns
            xv = xrow_v[0, pl.ds(i * 16, 16)]
            m = jnp.logical_and(xv >= own_lo, xv < own_hi)
            lv = xv - sbase
            bb = jax.lax.shift_left(i * 16 + iota, 13)
            ma = jnp.logical_and(m, lv < CHA)
            mb = jnp.logical_and(m, lv >= CHA)
            plsc.store_compressed(pka_v.at[pl.ds(nha, 16)], bb + lv, mask=ma)
            plsc.store_compressed(pkb_v.at[pl.ds(nhb, 16)], bb + lv - CHA,
                                  mask=mb)
            ca = plsc.all_reduce_population_count(ma)
            cb = plsc.all_reduce_population_count(mb)
            return (nha + ca[0], nhb + cb[0])

        nha, nhb = lax.fori_loop(0, NCH, scan_step,
                                 (jnp.int32(0), jnp.int32(0)))

        pltpu.make_async_copy(et_hbm.at[f, :, pl.ds(0, CHA)], slab_a,
                              sem_sa).wait()
        pltpu.make_async_copy(w_hbm.at[pl.ds(f, 1), pl.ds(0, WIN)],
                              wrow_v.at[:, pl.ds(0, WIN)], sem_w).wait()

        @pl.when(last)
        def _():
            pltpu.make_async_copy(wtail_hbm.at[pl.ds(f, 1), :],
                                  wrow_v.at[:, pl.ds(WIN, 128)],
                                  sem_w).wait()

        q_acc = hits(slab_a, pka_v, nha, 0, q_acc)

        @pl.when(f < F - 1)
        def _():
            fire_a(f + 1)

        pltpu.make_async_copy(et_hbm.at[f, :, pl.ds(0, CHB)],
                              slab_b.at[:, pl.ds(0, CHB)], sem_sb).wait()

        @pl.when(last)
        def _():
            pltpu.make_async_copy(etail_hbm.at[f],
                                  slab_b.at[:, pl.ds(CHB, 128)],
                                  sem_sb).wait()

        return hits(slab_b, pkb_v, nhb, CHA, q_acc)

    q_acc = lax.fori_loop(0, F, field_step, jnp.zeros((16,), jnp.float32))
    for h in range(8):
        part_v[0, pl.ds(h * 16, 16)] = jnp.zeros((16,), jnp.float32)
    part_v[0, pl.ds(0, 16)] = q_acc
    pltpu.sync_copy(part_v, parts_hbm.at[pl.ds(wid, 1), :])

    plsc.subcore_barrier()
    pltpu.sync_copy(stab_sh.at[pl.ds(row0, TROWS // NS), :],
                    tabs_hbm.at[c].at[pl.ds(row0, TROWS // NS), :])


@functools.partial(
    pl.kernel,
    out_type=(
        jax.ShapeDtypeStruct((NC, TROWS, 128), jnp.float32),
        jax.ShapeDtypeStruct((NW, 128), jnp.float32),
    ),
    mesh=plsc.VectorSubcoreMesh(core_axis_name="c", subcore_axis_name="s"),
    compiler_params=pltpu.CompilerParams(
        use_tc_tiling_on_sc=True, needs_layout_passes=False),
    scratch_types=[
        pltpu.VMEM((D, CHA), jnp.float32),
        pltpu.VMEM((D, CHA), jnp.float32),
        pltpu.VMEM((1, WIN + 128), jnp.float32),
        pltpu.VMEM((1, B), jnp.int32),
        pltpu.VMEM((B + 16,), jnp.int32),
        pltpu.VMEM((B + 16,), jnp.int32),
        pltpu.VMEM((96, 128), jnp.float32),
        pltpu.VMEM((1, 96), jnp.int32),
        pltpu.VMEM((1, 128), jnp.float32),
        pltpu.VMEM_SHARED((TROWS, 128), jnp.float32),
        pltpu.SemaphoreType.DMA,
        pltpu.SemaphoreType.DMA,
        pltpu.SemaphoreType.DMA,
        pltpu.SemaphoreType.DMA,
    ],
)
def _sc_fm(xt_hbm, et_hbm, w_hbm, etail_hbm, wtail_hbm, tabs_hbm, parts_hbm,
           slab_a, slab_b, wrow_v, xrow_v, pka_v, pkb_v, hit_v, b2d_v,
           part_v, stab_sh, sem_sa, sem_sb, sem_w, sem_x):
    _sc_body(xt_hbm, et_hbm, w_hbm, etail_hbm, wtail_hbm, tabs_hbm, parts_hbm,
             slab_a, slab_b, wrow_v, xrow_v, pka_v, pkb_v, hit_v, b2d_v,
             part_v, stab_sh, sem_sa, sem_sb, sem_w, sem_x)


def _tc_combine(tabs_ref, parts_ref, b_ref, out_ref):
    t = tabs_ref[0] + tabs_ref[1]
    sv = t[:B, :D]
    lin = jnp.sum(t[:B, D:2 * D], axis=1)
    q = jnp.sum(parts_ref[:, :D])
    inter = 0.5 * (jnp.sum(sv * sv) - q) + b_ref[0]
    out_ref[...] = lin + inter


def kernel(inputs, W_lin, b, E):
    x_t = inputs.astype(jnp.int32).T
    e_t = E.transpose(0, 2, 1)

    e_tail = jnp.pad(e_t[:, :, V - 32:], ((0, 0), (0, 0), (0, 96)))
    w_tail = jnp.pad(W_lin[:, V - 32:], ((0, 0), (0, 96)))
    tabs, parts = _sc_fm(x_t, e_t, W_lin, e_tail, w_tail)

    out = pl.pallas_call(
        _tc_combine,
        out_shape=jax.ShapeDtypeStruct((B,), jnp.float32),
    )(tabs, parts, b)
    return out[:, None]
